# baseline (speedup 1.0000x reference)
import functools

import jax
import jax.numpy as jnp
from jax import lax
from jax.experimental import pallas as pl
from jax.experimental.pallas import tpu as pltpu
from jax.experimental.pallas import tpu_sc as plsc

B = 16384
D = 32
NC = 2
NS = 16
NW = NC * NS
BPW = B // NW
CH = 128
NCH = BPW // CH

_sc_mesh = plsc.VectorSubcoreMesh(core_axis_name="c", subcore_axis_name="s")


@functools.partial(
    pl.kernel,
    mesh=_sc_mesh,
    compiler_params=pltpu.CompilerParams(use_tc_tiling_on_sc=False),
    out_type=[jax.ShapeDtypeStruct((B, D), jnp.float32)] * 2,
    scratch_types=[
        pltpu.VMEM((NCH, CH), jnp.int32),
        pltpu.VMEM((BPW, D), jnp.float32),
        pltpu.VMEM((BPW, D), jnp.float32),
        pltpu.SemaphoreType.DMA,
    ],
)
def _sc_movie_gather(mids, gme, mme, gm_o, mm_o, midx_v, gm_v, mm_v, sem):
    wid = lax.axis_index("s") * NC + lax.axis_index("c")
    base = wid * BPW
    for c in range(NCH):
        pltpu.sync_copy(mids.at[pl.ds(base + c * CH, CH)], midx_v.at[c])
    copies = []
    for c in range(NCH):
        row = pl.ds(c * CH, CH)
        copies.append(pltpu.async_copy(gme.at[midx_v.at[c]], gm_v.at[row], sem))
        copies.append(pltpu.async_copy(mme.at[midx_v.at[c]], mm_v.at[row], sem))
    for cp in copies:
        cp.wait()
    out_rows = pl.ds(base, BPW)
    pltpu.sync_copy(gm_v, gm_o.at[out_rows])
    pltpu.sync_copy(mm_v, mm_o.at[out_rows])


BLK = 2048


def _tc_gather_body(uids_s, gue, mue, gu_ref, mu_ref, semu, semm):
    def loop(j, _):
        r = uids_s[j]
        pltpu.make_async_copy(gue.at[r], gu_ref.at[j], semu).start()
        pltpu.make_async_copy(mue.at[r], mu_ref.at[j], semm).start()
        return 0
    lax.fori_loop(0, BLK, loop, 0, unroll=4)
    pltpu.make_async_copy(gue.at[pl.ds(0, BLK)], gu_ref, semu).wait()
    pltpu.make_async_copy(mue.at[pl.ds(0, BLK)], mu_ref, semm).wait()


def _tc_user_gather(user_ids, gue, mue):
    row_blk = pl.BlockSpec((BLK, D), lambda i: (i, 0))
    return pl.pallas_call(
        _tc_gather_body,
        grid=(B // BLK,),
        in_specs=[
            pl.BlockSpec((BLK,), lambda i: (i,), memory_space=pltpu.SMEM),
            pl.BlockSpec(memory_space=pl.ANY),
            pl.BlockSpec(memory_space=pl.ANY),
        ],
        out_specs=[row_blk, row_blk],
        out_shape=[jax.ShapeDtypeStruct((B, D), jnp.float32)] * 2,
        scratch_shapes=[
            pltpu.SemaphoreType.DMA,
            pltpu.SemaphoreType.DMA,
        ],
    )(user_ids, gue, mue)


def _tc_mlp_body(gu_r, mu_r, gm, mm, w1a, w1b, b1, w2, b2, wg, wh, bo, out_ref):
    f32 = jnp.float32
    gu = gu_r[...]
    mu = mu_r[...]
    h1 = jnp.dot(mu, w1a[...], preferred_element_type=f32)
    h1 = h1 + jnp.dot(mm[...], w1b[...], preferred_element_type=f32)
    h1 = jnp.maximum(h1 + b1[...], 0.0)
    h2 = jnp.maximum(jnp.dot(h1, w2[...], preferred_element_type=f32) + b2[...], 0.0)
    logit = jnp.dot(gu * gm[...], wg[...], preferred_element_type=f32)
    logit = logit + jnp.dot(h2, wh[...], preferred_element_type=f32)
    logit = logit + bo[...]
    out_ref[...] = 1.0 / (1.0 + jnp.exp(-logit))


def _tc_mlp(gu, mu, gm, mm, w1a, w1b, b1, W2, b2, wg, wh, bout):
    blk2 = lambda shape: pl.BlockSpec(shape, lambda i: (0, 0))
    blk1 = lambda shape: pl.BlockSpec(shape, lambda i: (0,))
    row_blk = pl.BlockSpec((BLK, D), lambda i: (i, 0))
    return pl.pallas_call(
        _tc_mlp_body,
        grid=(B // BLK,),
        in_specs=[
            row_blk, row_blk,
            row_blk, row_blk,
            blk2(w1a.shape), blk2(w1b.shape), blk1(b1.shape),
            blk2(W2.shape), blk1(b2.shape),
            blk2(wg.shape), blk2(wh.shape), blk1(bout.shape),
        ],
        out_specs=pl.BlockSpec((BLK, 1), lambda i: (i, 0)),
        out_shape=jax.ShapeDtypeStruct((B, 1), jnp.float32),
    )(gu, mu, gm, mm, w1a, w1b, b1, W2, b2, wg, wh, bout)


def kernel(user_ids, movie_ids, gmf_user_emb, gmf_movie_emb,
           mlp_user_emb, mlp_movie_emb, W1, b1, W2, b2, Wout, bout):
    gm, mm = _sc_movie_gather(movie_ids, gmf_movie_emb, mlp_movie_emb)
    gu, mu = _tc_user_gather(user_ids, gmf_user_emb, mlp_user_emb)
    out = _tc_mlp(gu, mu, gm, mm, W1[:D], W1[D:], b1, W2, b2,
                  Wout[:D], Wout[D:], bout)
    return out[:, 0]
